# baseline (speedup 1.0000x reference)
import functools

import jax
import jax.numpy as jnp
from jax import lax
from jax.experimental import pallas as pl
from jax.experimental.pallas import tpu as pltpu
from jax.experimental.pallas import tpu_sc as plsc

_N = 10000
_E = 320000
_D = 128
_NC = 2
_NS = 16
_NW = _NC * _NS
_NPAD = 10240
_RPT = _NPAD // _NS
_CH = 128
_CPT = 80
_EPAD = _NW * _CPT * _CH

_MESH = dict(core_axis_name="c", subcore_axis_name="s")


@functools.partial(
    pl.kernel,
    out_type=jax.ShapeDtypeStruct((_NC * _NPAD,), jnp.float32),
    mesh=plsc.VectorSubcoreMesh(**_MESH),
    scratch_types=[
        pltpu.VMEM((_CPT, _CH), jnp.int32),
        pltpu.VMEM((_CH,), jnp.float32),
        pltpu.VMEM((_RPT,), jnp.float32),
        pltpu.VMEM_SHARED((_NPAD,), jnp.float32),
        pltpu.SemaphoreType.DMA,
        pltpu.SemaphoreType.DMA,
    ],
)
def _deg_call(col_hbm, deg_hbm, coli2, ones_c, zbuf, deg_sh, psem, ssem):
    cid = lax.axis_index("c")
    sid = lax.axis_index("s")
    w = cid * _NS + sid
    pre = pltpu.async_copy(col_hbm.at[pl.ds(w * _CPT, _CPT)], coli2, psem)
    zeros16 = jnp.zeros((16,), jnp.float32)
    ones16 = jnp.ones((16,), jnp.float32)
    for j in range(_CH // 16):
        ones_c[pl.ds(j * 16, 16)] = ones16
    for j in range(_RPT // 16):
        zbuf[pl.ds(j * 16, 16)] = zeros16
    pltpu.sync_copy(zbuf, deg_sh.at[pl.ds(sid * _RPT, _RPT)])
    pre.wait()
    plsc.subcore_barrier()

    def fire(j, carry):
        pltpu.async_copy(ones_c, deg_sh.at[coli2.at[j]], ssem, add=True)
        return carry

    lax.fori_loop(0, _CPT, fire, 0)

    def drain(j, carry):
        pltpu.make_async_copy(ones_c, deg_sh.at[coli2.at[j]], ssem).wait()
        return carry

    lax.fori_loop(0, _CPT, drain, 0)
    plsc.subcore_barrier()
    pltpu.sync_copy(
        deg_sh.at[pl.ds(sid * _RPT, _RPT)],
        deg_hbm.at[pl.ds(cid * _NPAD + sid * _RPT, _RPT)],
    )


@functools.partial(
    pl.kernel,
    out_type=jax.ShapeDtypeStruct((_NC * _NPAD, _D), jnp.float32),
    mesh=plsc.VectorSubcoreMesh(**_MESH),
    scratch_types=[
        pltpu.VMEM((_CPT, _CH), jnp.int32),
        pltpu.VMEM((_CH,), jnp.int32),
        pltpu.VMEM((_CH,), jnp.int32),
        pltpu.VMEM((_CH, _D), jnp.float32),
        pltpu.VMEM((_CH, _D), jnp.float32),
        pltpu.VMEM((16, _D), jnp.float32),
        pltpu.VMEM_SHARED((_NPAD, _D), jnp.float32),
        pltpu.SemaphoreType.DMA,
        pltpu.SemaphoreType.DMA,
        pltpu.SemaphoreType.DMA,
        pltpu.SemaphoreType.DMA,
    ],
)
def _msg_call(row_hbm, col_hbm, g_hbm, out_hbm,
              rowi2, colia, colib, bufa, bufb, zb, acc_sh,
              psem, zsem, sema, semb):
    cid = lax.axis_index("c")
    sid = lax.axis_index("s")
    w = cid * _NS + sid
    ebase = w * (_CPT * _CH)
    prer = pltpu.async_copy(row_hbm.at[pl.ds(w * _CPT, _CPT)], rowi2, psem)
    zeros16 = jnp.zeros((16,), jnp.float32)
    for i in range(16):
        for j in range(_D // 16):
            zb[i, pl.ds(j * 16, 16)] = zeros16

    def zfire(t, carry):
        pltpu.async_copy(zb, acc_sh.at[pl.ds(sid * _RPT + t * 16, 16)], zsem)
        return carry

    lax.fori_loop(0, _RPT // 16, zfire, 0)

    def zdrain(t, carry):
        pltpu.make_async_copy(
            zb, acc_sh.at[pl.ds(sid * _RPT + t * 16, 16)], zsem).wait()
        return carry

    lax.fori_loop(0, _RPT // 16, zdrain, 0)
    prer.wait()

    def cref(j):
        return col_hbm.at[pl.ds(ebase + j * _CH, _CH)]

    def start(j, cbuf, gbuf, sem):
        pltpu.async_copy(cref(j), cbuf, sem)
        pltpu.async_copy(g_hbm.at[rowi2.at[j]], gbuf, sem)

    def finish(j, cbuf, gbuf, sem):
        pltpu.make_async_copy(cref(j), cbuf, sem).wait()
        pltpu.make_async_copy(g_hbm.at[rowi2.at[j]], gbuf, sem).wait()
        pltpu.sync_copy(gbuf, acc_sh.at[cbuf], add=True)

    start(0, colia, bufa, sema)
    start(1, colib, bufb, semb)
    plsc.subcore_barrier()

    def body(k, carry):
        j = 2 * k
        finish(j, colia, bufa, sema)
        start(j + 2, colia, bufa, sema)
        finish(j + 1, colib, bufb, semb)
        start(j + 3, colib, bufb, semb)
        return carry

    lax.fori_loop(0, (_CPT - 2) // 2, body, 0)
    finish(_CPT - 2, colia, bufa, sema)
    finish(_CPT - 1, colib, bufb, semb)
    plsc.subcore_barrier()
    pltpu.sync_copy(
        acc_sh.at[pl.ds(sid * _RPT, _RPT)],
        out_hbm.at[pl.ds(cid * _NPAD + sid * _RPT, _RPT)],
    )


def _lin_body(x_ref, w_ref, degp_ref, g_ref, dis_ref):
    deg = degp_ref[0, :] + degp_ref[1, :]
    dis = jnp.where(deg > 0.0, lax.rsqrt(deg), 0.0)
    dis_ref[...] = dis
    h = lax.dot_general(
        x_ref[...], w_ref[...], (((1,), (1,)), ((), ())),
        preferred_element_type=jnp.float32,
    )
    g_ref[...] = h * dis[:_N][:, None]


def _bn_body(s_ref, dis_ref, b_ref, g_ref, be_ref, x_ref, o_ref):
    s = s_ref[pl.ds(0, _N), :] + s_ref[pl.ds(_NPAD, _N), :]
    pre = s * dis_ref[pl.ds(0, _N)][:, None] + b_ref[...][None, :]
    mean = jnp.mean(pre, axis=0)
    cen = pre - mean[None, :]
    var = jnp.mean(cen * cen, axis=0)
    o_ref[...] = (
        cen * (lax.rsqrt(var + 1e-5) * g_ref[...])[None, :]
        + be_ref[...][None, :] + x_ref[...]
    )


def kernel(x, edge_index, bit_sum, W, bias, bn_gamma, bn_beta):
    npad = _EPAD - _E
    ar = jnp.arange(npad, dtype=jnp.int32)
    row2d = jnp.concatenate([edge_index[0], ar % _N]).reshape(-1, _CH)
    col2d = jnp.concatenate(
        [edge_index[1], _N + ar % (_NPAD - _N)]).reshape(-1, _CH)
    degf = _deg_call(col2d)
    g, dis = pl.pallas_call(
        _lin_body,
        out_shape=[
            jax.ShapeDtypeStruct((_N, _D), jnp.float32),
            jax.ShapeDtypeStruct((_NPAD,), jnp.float32),
        ],
    )(x, W, degf.reshape(_NC, _NPAD))
    sflat = _msg_call(row2d, col2d.reshape(-1), g)
    out = pl.pallas_call(
        _bn_body,
        out_shape=jax.ShapeDtypeStruct((_N, _D), jnp.float32),
    )(sflat, dis, bias, bn_gamma, bn_beta, x)
    return (out, jnp.asarray(0, dtype=jnp.int32))


# speedup vs baseline: 1.0843x; 1.0843x over previous
import functools

import jax
import jax.numpy as jnp
from jax import lax
from jax.experimental import pallas as pl
from jax.experimental.pallas import tpu as pltpu
from jax.experimental.pallas import tpu_sc as plsc

_N = 10000
_E = 320000
_D = 128
_NC = 2
_NS = 16
_NW = _NC * _NS
_NPAD = 10240
_RPT = _NPAD // _NS
_CH = 128
_CPT = 80
_EPT = _CPT * _CH
_LASTC = (_E - (_NW - 1) * _EPT) // _CH
_RING = 16

_MESH = dict(core_axis_name="c", subcore_axis_name="s")


@functools.partial(
    pl.kernel,
    out_type=jax.ShapeDtypeStruct((_NC * _NPAD,), jnp.float32),
    mesh=plsc.VectorSubcoreMesh(**_MESH),
    scratch_types=[
        pltpu.VMEM((_RING, _CH), jnp.int32),
        pltpu.VMEM((_CH,), jnp.float32),
        pltpu.VMEM((_RPT,), jnp.float32),
        pltpu.VMEM_SHARED((_NPAD,), jnp.float32),
        pltpu.SemaphoreType.DMA,
        pltpu.SemaphoreType.DMA,
    ],
)
def _deg_call(edge_hbm, deg_hbm, colb, ones_c, zbuf, deg_sh, lsem, ssem):
    cid = lax.axis_index("c")
    sid = lax.axis_index("s")
    w = cid * _NS + sid
    nch = jnp.where(w == _NW - 1, _LASTC, _CPT)
    ebase = w * _EPT

    def cref(j):
        return edge_hbm.at[1, pl.ds(ebase + j * _CH, _CH)]

    for j in range(_RING // 2):
        pltpu.async_copy(cref(j), colb.at[j], lsem)
    zeros16 = jnp.zeros((16,), jnp.float32)
    ones16 = jnp.ones((16,), jnp.float32)
    for j in range(_CH // 16):
        ones_c[pl.ds(j * 16, 16)] = ones16
    for j in range(_RPT // 16):
        zbuf[pl.ds(j * 16, 16)] = zeros16
    pltpu.sync_copy(zbuf, deg_sh.at[pl.ds(sid * _RPT, _RPT)])
    plsc.subcore_barrier()

    def body(j, carry):
        b = lax.rem(j, _RING)

        @pl.when(j >= _RING // 2)
        def _():
            pltpu.make_async_copy(
                ones_c, deg_sh.at[colb.at[b]], ssem).wait()

        pltpu.make_async_copy(cref(j), colb.at[b], lsem).wait()
        pltpu.async_copy(ones_c, deg_sh.at[colb.at[b]], ssem, add=True)

        @pl.when(j + _RING // 2 < nch)
        def _():
            bn = lax.rem(j + _RING // 2, _RING)
            pltpu.async_copy(cref(j + _RING // 2), colb.at[bn], lsem)

        return carry

    lax.fori_loop(0, nch, body, 0)

    def drain(j, carry):
        pltpu.make_async_copy(ones_c, deg_sh.at[colb.at[0]], ssem).wait()
        return carry

    lax.fori_loop(0, _RING // 2, drain, 0)
    plsc.subcore_barrier()
    pltpu.sync_copy(
        deg_sh.at[pl.ds(sid * _RPT, _RPT)],
        deg_hbm.at[pl.ds(cid * _NPAD + sid * _RPT, _RPT)],
    )


@functools.partial(
    pl.kernel,
    out_type=jax.ShapeDtypeStruct((_NC * _NPAD, _D), jnp.float32),
    mesh=plsc.VectorSubcoreMesh(**_MESH),
    scratch_types=[
        pltpu.VMEM((_EPT,), jnp.int32),
        pltpu.VMEM((_CH,), jnp.int32),
        pltpu.VMEM((_CH,), jnp.int32),
        pltpu.VMEM((_CH, _D), jnp.float32),
        pltpu.VMEM((_CH, _D), jnp.float32),
        pltpu.VMEM((16, _D), jnp.float32),
        pltpu.VMEM_SHARED((_NPAD, _D), jnp.float32),
        pltpu.SemaphoreType.DMA,
        pltpu.SemaphoreType.DMA,
        pltpu.SemaphoreType.DMA,
        pltpu.SemaphoreType.DMA,
    ],
)
def _msg_call(edge_hbm, g_hbm, out_hbm,
              rowi, colia, colib, bufa, bufb, zb, acc_sh,
              psem, zsem, sema, semb):
    cid = lax.axis_index("c")
    sid = lax.axis_index("s")
    w = cid * _NS + sid
    nch = jnp.where(w == _NW - 1, _LASTC, _CPT)
    ebase = w * _EPT
    nlast = _LASTC * _CH

    @pl.when(w < _NW - 1)
    def _():
        pltpu.async_copy(edge_hbm.at[0, pl.ds(ebase, _EPT)], rowi, psem)

    @pl.when(w == _NW - 1)
    def _():
        pltpu.async_copy(
            edge_hbm.at[0, pl.ds((_NW - 1) * _EPT, nlast)],
            rowi.at[pl.ds(0, nlast)], psem)

    zeros16 = jnp.zeros((16,), jnp.float32)
    for i in range(16):
        for j in range(_D // 16):
            zb[i, pl.ds(j * 16, 16)] = zeros16

    def zfire(t, carry):
        pltpu.async_copy(zb, acc_sh.at[pl.ds(sid * _RPT + t * 16, 16)], zsem)
        return carry

    lax.fori_loop(0, _RPT // 16, zfire, 0)

    def zdrain(t, carry):
        pltpu.make_async_copy(
            zb, acc_sh.at[pl.ds(sid * _RPT + t * 16, 16)], zsem).wait()
        return carry

    lax.fori_loop(0, _RPT // 16, zdrain, 0)

    @pl.when(w < _NW - 1)
    def _():
        pltpu.make_async_copy(
            edge_hbm.at[0, pl.ds(ebase, _EPT)], rowi, psem).wait()

    @pl.when(w == _NW - 1)
    def _():
        pltpu.make_async_copy(
            edge_hbm.at[0, pl.ds((_NW - 1) * _EPT, nlast)],
            rowi.at[pl.ds(0, nlast)], psem).wait()

    def cref(j):
        return edge_hbm.at[1, pl.ds(ebase + j * _CH, _CH)]

    def start(j, cbuf, gbuf, sem):
        pltpu.async_copy(cref(j), cbuf, sem)
        pltpu.async_copy(g_hbm.at[rowi.at[pl.ds(j * _CH, _CH)]], gbuf, sem)

    def finish(j, cbuf, gbuf, sem):
        pltpu.make_async_copy(cref(j), cbuf, sem).wait()
        pltpu.make_async_copy(
            g_hbm.at[rowi.at[pl.ds(j * _CH, _CH)]], gbuf, sem).wait()
        pltpu.sync_copy(gbuf, acc_sh.at[cbuf], add=True)

    start(0, colia, bufa, sema)
    start(1, colib, bufb, semb)
    plsc.subcore_barrier()

    def body(k, carry):
        j = 2 * k
        finish(j, colia, bufa, sema)
        start(j + 2, colia, bufa, sema)
        finish(j + 1, colib, bufb, semb)
        start(j + 3, colib, bufb, semb)
        return carry

    lax.fori_loop(0, (nch - 2) // 2, body, 0)
    finish(nch - 2, colia, bufa, sema)
    finish(nch - 1, colib, bufb, semb)
    plsc.subcore_barrier()
    pltpu.sync_copy(
        acc_sh.at[pl.ds(sid * _RPT, _RPT)],
        out_hbm.at[pl.ds(cid * _NPAD + sid * _RPT, _RPT)],
    )


def _lin_body(x_ref, w_ref, degf_ref, g_ref, dis_ref):
    deg = degf_ref[pl.ds(0, _NPAD)] + degf_ref[pl.ds(_NPAD, _NPAD)]
    dis = jnp.where(deg > 0.0, lax.rsqrt(deg), 0.0)
    dis_ref[...] = dis
    h = lax.dot_general(
        x_ref[...], w_ref[...], (((1,), (1,)), ((), ())),
        preferred_element_type=jnp.float32,
    )
    g_ref[...] = h * dis[:_N][:, None]


def _bn_body(s_ref, dis_ref, b_ref, g_ref, be_ref, x_ref, o_ref):
    s = s_ref[pl.ds(0, _N), :] + s_ref[pl.ds(_NPAD, _N), :]
    pre = s * dis_ref[pl.ds(0, _N)][:, None] + b_ref[...][None, :]
    mean = jnp.mean(pre, axis=0)
    cen = pre - mean[None, :]
    var = jnp.mean(cen * cen, axis=0)
    o_ref[...] = (
        cen * (lax.rsqrt(var + 1e-5) * g_ref[...])[None, :]
        + be_ref[...][None, :] + x_ref[...]
    )


def kernel(x, edge_index, bit_sum, W, bias, bn_gamma, bn_beta):
    degf = _deg_call(edge_index)
    g, dis = pl.pallas_call(
        _lin_body,
        out_shape=[
            jax.ShapeDtypeStruct((_N, _D), jnp.float32),
            jax.ShapeDtypeStruct((_NPAD,), jnp.float32),
        ],
    )(x, W, degf)
    sflat = _msg_call(edge_index, g)
    out = pl.pallas_call(
        _bn_body,
        out_shape=jax.ShapeDtypeStruct((_N, _D), jnp.float32),
    )(sflat, dis, bias, bn_gamma, bn_beta, x)
    return (out, jnp.asarray(0, dtype=jnp.int32))


# speedup vs baseline: 1.0858x; 1.0013x over previous
import functools

import jax
import jax.numpy as jnp
from jax import lax
from jax.experimental import pallas as pl
from jax.experimental.pallas import tpu as pltpu
from jax.experimental.pallas import tpu_sc as plsc

_N = 10000
_E = 320000
_D = 128
_NC = 2
_NS = 16
_NW = _NC * _NS
_NPAD = 10240
_RPT = _NPAD // _NS
_CH = 128
_CPT = 80
_EPT = _CPT * _CH
_LASTC = (_E - (_NW - 1) * _EPT) // _CH
_RING = 16

_MESH = dict(core_axis_name="c", subcore_axis_name="s")


@functools.partial(
    pl.kernel,
    out_type=jax.ShapeDtypeStruct((_NC * _NPAD,), jnp.float32),
    mesh=plsc.VectorSubcoreMesh(**_MESH),
    compiler_params=pltpu.CompilerParams(use_tc_tiling_on_sc=True),
    scratch_types=[
        pltpu.VMEM((_RING, _CH), jnp.int32),
        pltpu.VMEM((_CH,), jnp.float32),
        pltpu.VMEM((_RPT,), jnp.float32),
        pltpu.VMEM_SHARED((_NPAD,), jnp.float32),
        pltpu.SemaphoreType.DMA,
        pltpu.SemaphoreType.DMA,
    ],
)
def _deg_call(edge_hbm, deg_hbm, colb, ones_c, zbuf, deg_sh, lsem, ssem):
    cid = lax.axis_index("c")
    sid = lax.axis_index("s")
    w = cid * _NS + sid
    nch = jnp.where(w == _NW - 1, _LASTC, _CPT)
    ebase = w * _EPT

    def cref(j):
        return edge_hbm.at[1, pl.ds(ebase + j * _CH, _CH)]

    for j in range(_RING // 2):
        pltpu.async_copy(cref(j), colb.at[j], lsem)
    zeros16 = jnp.zeros((16,), jnp.float32)
    ones16 = jnp.ones((16,), jnp.float32)
    for j in range(_CH // 16):
        ones_c[pl.ds(j * 16, 16)] = ones16
    for j in range(_RPT // 16):
        zbuf[pl.ds(j * 16, 16)] = zeros16
    pltpu.sync_copy(zbuf, deg_sh.at[pl.ds(sid * _RPT, _RPT)])
    plsc.subcore_barrier()

    def body(j, carry):
        b = lax.rem(j, _RING)

        @pl.when(j >= _RING // 2)
        def _():
            pltpu.make_async_copy(
                ones_c, deg_sh.at[colb.at[b]], ssem).wait()

        pltpu.make_async_copy(cref(j), colb.at[b], lsem).wait()
        pltpu.async_copy(ones_c, deg_sh.at[colb.at[b]], ssem, add=True)

        @pl.when(j + _RING // 2 < nch)
        def _():
            bn = lax.rem(j + _RING // 2, _RING)
            pltpu.async_copy(cref(j + _RING // 2), colb.at[bn], lsem)

        return carry

    lax.fori_loop(0, nch, body, 0)

    def drain(j, carry):
        pltpu.make_async_copy(ones_c, deg_sh.at[colb.at[0]], ssem).wait()
        return carry

    lax.fori_loop(0, _RING // 2, drain, 0)
    plsc.subcore_barrier()
    pltpu.sync_copy(
        deg_sh.at[pl.ds(sid * _RPT, _RPT)],
        deg_hbm.at[pl.ds(cid * _NPAD + sid * _RPT, _RPT)],
    )


@functools.partial(
    pl.kernel,
    out_type=jax.ShapeDtypeStruct((_NC * _NPAD, _D), jnp.float32),
    mesh=plsc.VectorSubcoreMesh(**_MESH),
    compiler_params=pltpu.CompilerParams(use_tc_tiling_on_sc=True),
    scratch_types=[
        pltpu.VMEM((_EPT,), jnp.int32),
        pltpu.VMEM((_CH,), jnp.int32),
        pltpu.VMEM((_CH,), jnp.int32),
        pltpu.VMEM((_CH, _D), jnp.float32),
        pltpu.VMEM((_CH, _D), jnp.float32),
        pltpu.VMEM((16, _D), jnp.float32),
        pltpu.VMEM_SHARED((_NPAD, _D), jnp.float32),
        pltpu.SemaphoreType.DMA,
        pltpu.SemaphoreType.DMA,
        pltpu.SemaphoreType.DMA,
        pltpu.SemaphoreType.DMA,
    ],
)
def _msg_call(edge_hbm, g_hbm, out_hbm,
              rowi, colia, colib, bufa, bufb, zb, acc_sh,
              psem, zsem, sema, semb):
    cid = lax.axis_index("c")
    sid = lax.axis_index("s")
    w = cid * _NS + sid
    nch = jnp.where(w == _NW - 1, _LASTC, _CPT)
    ebase = w * _EPT
    nlast = _LASTC * _CH

    @pl.when(w < _NW - 1)
    def _():
        pltpu.async_copy(edge_hbm.at[0, pl.ds(ebase, _EPT)], rowi, psem)

    @pl.when(w == _NW - 1)
    def _():
        pltpu.async_copy(
            edge_hbm.at[0, pl.ds((_NW - 1) * _EPT, nlast)],
            rowi.at[pl.ds(0, nlast)], psem)

    zeros16 = jnp.zeros((16,), jnp.float32)
    for i in range(16):
        for j in range(_D // 16):
            zb[i, pl.ds(j * 16, 16)] = zeros16

    def zfire(t, carry):
        pltpu.async_copy(zb, acc_sh.at[pl.ds(sid * _RPT + t * 16, 16)], zsem)
        return carry

    lax.fori_loop(0, _RPT // 16, zfire, 0)

    def zdrain(t, carry):
        pltpu.make_async_copy(
            zb, acc_sh.at[pl.ds(sid * _RPT + t * 16, 16)], zsem).wait()
        return carry

    lax.fori_loop(0, _RPT // 16, zdrain, 0)

    @pl.when(w < _NW - 1)
    def _():
        pltpu.make_async_copy(
            edge_hbm.at[0, pl.ds(ebase, _EPT)], rowi, psem).wait()

    @pl.when(w == _NW - 1)
    def _():
        pltpu.make_async_copy(
            edge_hbm.at[0, pl.ds((_NW - 1) * _EPT, nlast)],
            rowi.at[pl.ds(0, nlast)], psem).wait()

    def cref(j):
        return edge_hbm.at[1, pl.ds(ebase + j * _CH, _CH)]

    def start(j, cbuf, gbuf, sem):
        pltpu.async_copy(cref(j), cbuf, sem)
        pltpu.async_copy(g_hbm.at[rowi.at[pl.ds(j * _CH, _CH)]], gbuf, sem)

    def finish(j, cbuf, gbuf, sem):
        pltpu.make_async_copy(cref(j), cbuf, sem).wait()
        pltpu.make_async_copy(
            g_hbm.at[rowi.at[pl.ds(j * _CH, _CH)]], gbuf, sem).wait()
        pltpu.sync_copy(gbuf, acc_sh.at[cbuf], add=True)

    start(0, colia, bufa, sema)
    start(1, colib, bufb, semb)
    plsc.subcore_barrier()

    def body(k, carry):
        j = 2 * k
        finish(j, colia, bufa, sema)
        start(j + 2, colia, bufa, sema)
        finish(j + 1, colib, bufb, semb)
        start(j + 3, colib, bufb, semb)
        return carry

    lax.fori_loop(0, (nch - 2) // 2, body, 0)
    finish(nch - 2, colia, bufa, sema)
    finish(nch - 1, colib, bufb, semb)
    plsc.subcore_barrier()
    pltpu.sync_copy(
        acc_sh.at[pl.ds(sid * _RPT, _RPT)],
        out_hbm.at[pl.ds(cid * _NPAD + sid * _RPT, _RPT)],
    )


def _lin_body(x_ref, w_ref, degf_ref, g_ref, dis_ref):
    deg = degf_ref[pl.ds(0, _NPAD)] + degf_ref[pl.ds(_NPAD, _NPAD)]
    dis = jnp.where(deg > 0.0, lax.rsqrt(deg), 0.0)
    dis_ref[...] = dis
    h = lax.dot_general(
        x_ref[...], w_ref[...], (((1,), (1,)), ((), ())),
        preferred_element_type=jnp.float32,
    )
    g_ref[...] = h * dis[:_N][:, None]


def _bn_body(s_ref, dis_ref, b_ref, g_ref, be_ref, x_ref, o_ref):
    s = s_ref[pl.ds(0, _N), :] + s_ref[pl.ds(_NPAD, _N), :]
    pre = s * dis_ref[pl.ds(0, _N)][:, None] + b_ref[...][None, :]
    mean = jnp.mean(pre, axis=0)
    cen = pre - mean[None, :]
    var = jnp.mean(cen * cen, axis=0)
    o_ref[...] = (
        cen * (lax.rsqrt(var + 1e-5) * g_ref[...])[None, :]
        + be_ref[...][None, :] + x_ref[...]
    )


def kernel(x, edge_index, bit_sum, W, bias, bn_gamma, bn_beta):
    degf = _deg_call(edge_index)
    g, dis = pl.pallas_call(
        _lin_body,
        out_shape=[
            jax.ShapeDtypeStruct((_N, _D), jnp.float32),
            jax.ShapeDtypeStruct((_NPAD,), jnp.float32),
        ],
    )(x, W, degf)
    sflat = _msg_call(edge_index, g)
    out = pl.pallas_call(
        _bn_body,
        out_shape=jax.ShapeDtypeStruct((_N, _D), jnp.float32),
    )(sflat, dis, bias, bn_gamma, bn_beta, x)
    return (out, jnp.asarray(0, dtype=jnp.int32))
